# Initial kernel scaffold; baseline (speedup 1.0000x reference)
import jax
import jax.numpy as jnp
from jax.experimental import pallas as pl


def kernel(x, edge_index, W_in, b_in, Wg, bg, gamma, beta, Wo1, bo1, Wo2, bo2):
    raise NotImplementedError("write your pallas kernel here")


# speedup vs baseline: 1.1880x; 1.1880x over previous
import jax
import jax.numpy as jnp
import numpy as np
from jax import lax
from jax.experimental import pallas as pl
from jax.experimental.pallas import tpu as pltpu
from jax.experimental.pallas import tpu_sc as plsc

N = 10000
E = 320000
H = 128
T = 13
L = 6
EPS = 1e-5
BN_SCALE = 1.0 / np.sqrt(1.0 + EPS)

NC = 2
NS = 16
NW = NC * NS
IPC = 128
CPT = 80
EPT = IPC * CPT
EPAD = EPT * NW
NP = 10240
RPT = NP // NS

ROWB = 1024
GRID = NP // ROWB


def _mesh():
    return plsc.VectorSubcoreMesh(
        core_axis_name="c", subcore_axis_name="s", num_cores=NC, num_subcores=NS
    )


def _spmm_body(ms_hbm, row_hbm, col_hbm, zeros_hbm, out_hbm,
               ridx, cidx, rows_v, acc, sem):
    c = lax.axis_index("c")
    s = lax.axis_index("s")
    pltpu.sync_copy(zeros_hbm.at[pl.ds(s * RPT, RPT)],
                    acc.at[pl.ds(s * RPT, RPT)])
    tb = (c * NS + s) * CPT
    pltpu.sync_copy(row_hbm.at[pl.ds(tb, CPT)], ridx)
    pltpu.sync_copy(col_hbm.at[pl.ds(tb, CPT)], cidx)
    plsc.subcore_barrier()

    def chunk(j, _):
        pltpu.async_copy(ms_hbm.at[ridx.at[j]], rows_v, sem).wait()
        pltpu.sync_copy(rows_v, acc.at[cidx.at[j]], add=True)
        return 0

    lax.fori_loop(0, CPT, chunk, 0)
    plsc.subcore_barrier()
    pltpu.sync_copy(acc.at[pl.ds(s * RPT, RPT)],
                    out_hbm.at[pl.ds(c * NP + s * RPT, RPT)])


def _spmm_call(ms, row2d, col2d, zeros128):
    k = pl.kernel(
        _spmm_body,
        out_type=jax.ShapeDtypeStruct((2 * NP, H), jnp.float32),
        mesh=_mesh(),
        scratch_types=[
            pltpu.VMEM((CPT, IPC), jnp.int32),
            pltpu.VMEM((CPT, IPC), jnp.int32),
            pltpu.VMEM((IPC, H), jnp.float32),
            pltpu.VMEM_SHARED((NP, H), jnp.float32),
            pltpu.SemaphoreType.DMA,
        ],
    )
    return k(ms, row2d, col2d, zeros128)


def _t1_body(x_ref, win_ref, bin_ref, wg0_ref, d0_ref, d1_ref,
             h0_ref, ms0_ref, dinv_ref):
    deg = 1.0 + d0_ref[:, 0:1] + d1_ref[:, 0:1]
    dv = lax.rsqrt(deg)
    h0 = jnp.dot(x_ref[...], win_ref[...],
                 preferred_element_type=jnp.float32) + bin_ref[...]
    h0_ref[...] = h0
    ms0_ref[...] = jnp.dot(h0, wg0_ref[...],
                           preferred_element_type=jnp.float32) * dv
    dinv_ref[...] = dv


def _t1_call(xp, W_in, b_in2, Wg0, degp):
    return pl.pallas_call(
        _t1_body,
        grid=(GRID,),
        in_specs=[
            pl.BlockSpec((ROWB, H), lambda i: (i, 0)),
            pl.BlockSpec((H, H), lambda i: (0, 0)),
            pl.BlockSpec((1, H), lambda i: (0, 0)),
            pl.BlockSpec((H, H), lambda i: (0, 0)),
            pl.BlockSpec((ROWB, H), lambda i: (i, 0)),
            pl.BlockSpec((ROWB, H), lambda i: (GRID + i, 0)),
        ],
        out_specs=[
            pl.BlockSpec((ROWB, H), lambda i: (i, 0)),
            pl.BlockSpec((ROWB, H), lambda i: (i, 0)),
            pl.BlockSpec((ROWB, 1), lambda i: (i, 0)),
        ],
        out_shape=[
            jax.ShapeDtypeStruct((NP, H), jnp.float32),
            jax.ShapeDtypeStruct((NP, H), jnp.float32),
            jax.ShapeDtypeStruct((NP, 1), jnp.float32),
        ],
    )(xp, W_in, b_in2, Wg0, degp, degp)


def _t2_body(p0_ref, p1_ref, ms_ref, h_ref, dinv_ref, bg_ref, gm_ref, bt_ref,
             wn_ref, hn_ref, msn_ref):
    dv = dinv_ref[...]
    agg = dv * (p0_ref[...] + p1_ref[...] + ms_ref[...]) + bg_ref[...]
    hb = agg * BN_SCALE * gm_ref[...] + bt_ref[...]
    hn = jnp.maximum(hb, 0.0) + h_ref[...]
    hn_ref[...] = hn
    msn_ref[...] = jnp.dot(hn, wn_ref[...],
                           preferred_element_type=jnp.float32) * dv


def _t2_call(P, ms, h, dinv, bg2, gm2, bt2, Wnext):
    return pl.pallas_call(
        _t2_body,
        grid=(GRID,),
        in_specs=[
            pl.BlockSpec((ROWB, H), lambda i: (i, 0)),
            pl.BlockSpec((ROWB, H), lambda i: (GRID + i, 0)),
            pl.BlockSpec((ROWB, H), lambda i: (i, 0)),
            pl.BlockSpec((ROWB, H), lambda i: (i, 0)),
            pl.BlockSpec((ROWB, 1), lambda i: (i, 0)),
            pl.BlockSpec((1, H), lambda i: (0, 0)),
            pl.BlockSpec((1, H), lambda i: (0, 0)),
            pl.BlockSpec((1, H), lambda i: (0, 0)),
            pl.BlockSpec((H, H), lambda i: (0, 0)),
        ],
        out_specs=[
            pl.BlockSpec((ROWB, H), lambda i: (i, 0)),
            pl.BlockSpec((ROWB, H), lambda i: (i, 0)),
        ],
        out_shape=[
            jax.ShapeDtypeStruct((NP, H), jnp.float32),
            jax.ShapeDtypeStruct((NP, H), jnp.float32),
        ],
    )(P, P, ms, h, dinv, bg2, gm2, bt2, Wnext)


def _t3_body(p0_ref, p1_ref, ms_ref, h_ref, dinv_ref, bg_ref, gm_ref, bt_ref,
             wo1_ref, bo1_ref, wo2_ref, bo2_ref, out_ref, acc):
    i = pl.program_id(0)
    dv = dinv_ref[...]
    agg = dv * (p0_ref[...] + p1_ref[...] + ms_ref[...]) + bg_ref[...]
    hb = agg * BN_SCALE * gm_ref[...] + bt_ref[...]
    hn = jnp.maximum(hb, 0.0) + h_ref[...]
    rid = i * ROWB + lax.broadcasted_iota(jnp.int32, (ROWB, 1), 0)
    hn = jnp.where(rid < N, hn, 0.0)
    bsum = jnp.sum(hn, axis=0, keepdims=True)

    @pl.when(i == 0)
    def _():
        acc[...] = bsum

    @pl.when(i > 0)
    def _():
        acc[...] = acc[...] + bsum

    @pl.when(i == pl.num_programs(0) - 1)
    def _():
        pooled = acc[...] * (1.0 / N)
        o = jnp.maximum(
            jnp.dot(pooled, wo1_ref[...],
                    preferred_element_type=jnp.float32) + bo1_ref[...], 0.0)
        out_ref[...] = jnp.dot(o, wo2_ref[...],
                               preferred_element_type=jnp.float32) + bo2_ref[...]


def _t3_call(P, ms, h, dinv, bg2, gm2, bt2, Wo1p, bo1p, Wo2p, bo2p):
    return pl.pallas_call(
        _t3_body,
        grid=(GRID,),
        in_specs=[
            pl.BlockSpec((ROWB, H), lambda i: (i, 0)),
            pl.BlockSpec((ROWB, H), lambda i: (GRID + i, 0)),
            pl.BlockSpec((ROWB, H), lambda i: (i, 0)),
            pl.BlockSpec((ROWB, H), lambda i: (i, 0)),
            pl.BlockSpec((ROWB, 1), lambda i: (i, 0)),
            pl.BlockSpec((1, H), lambda i: (0, 0)),
            pl.BlockSpec((1, H), lambda i: (0, 0)),
            pl.BlockSpec((1, H), lambda i: (0, 0)),
            pl.BlockSpec((H, H), lambda i: (0, 0)),
            pl.BlockSpec((1, H), lambda i: (0, 0)),
            pl.BlockSpec((H, H), lambda i: (0, 0)),
            pl.BlockSpec((1, H), lambda i: (0, 0)),
        ],
        out_specs=pl.BlockSpec((1, H), lambda i: (0, 0)),
        out_shape=jax.ShapeDtypeStruct((1, H), jnp.float32),
        scratch_shapes=[pltpu.VMEM((1, H), jnp.float32)],
    )(P, P, ms, h, dinv, bg2, gm2, bt2, Wo1p, bo1p, Wo2p, bo2p)


def kernel(x, edge_index, W_in, b_in, Wg, bg, gamma, beta, Wo1, bo1, Wo2, bo2):
    pad = EPAD - E
    row = jnp.concatenate([edge_index[0], jnp.zeros((pad,), jnp.int32)])
    col = jnp.concatenate([edge_index[1], jnp.full((pad,), N, jnp.int32)])
    row2d = row.reshape(EPAD // IPC, IPC)
    col2d = col.reshape(EPAD // IPC, IPC)
    zeros128 = jnp.zeros((NP, H), jnp.float32)
    ones128 = jnp.ones((NP, H), jnp.float32)
    zrow2d = jnp.zeros_like(row2d)
    xp = jnp.zeros((NP, H), jnp.float32).at[:N].set(x)

    b_in2 = b_in.reshape(1, H)
    bg2 = bg.reshape(L, 1, H)
    gm2 = gamma.reshape(L, 1, H)
    bt2 = beta.reshape(L, 1, H)
    Wo1p = jnp.zeros((H, H), jnp.float32).at[:, : H // 2].set(Wo1)
    bo1p = jnp.zeros((1, H), jnp.float32).at[:, : H // 2].set(bo1)
    Wo2p = jnp.zeros((H, H), jnp.float32).at[: H // 2, :T].set(Wo2)
    bo2p = jnp.zeros((1, H), jnp.float32).at[:, :T].set(bo2)

    degp = _spmm_call(ones128, zrow2d, col2d, zeros128)
    h, ms, dinv = _t1_call(xp, W_in, b_in2, Wg[0], degp)
    for l in range(L - 1):
        P = _spmm_call(ms, row2d, col2d, zeros128)
        h, ms = _t2_call(P, ms, h, dinv, bg2[l], gm2[l], bt2[l], Wg[l + 1])
    P = _spmm_call(ms, row2d, col2d, zeros128)
    out = _t3_call(P, ms, h, dinv, bg2[L - 1], gm2[L - 1], bt2[L - 1],
                   Wo1p, bo1p, Wo2p, bo2p)
    return out[:, :T]


# speedup vs baseline: 5.3489x; 4.5025x over previous
import jax
import jax.numpy as jnp
import numpy as np
from jax import lax
from jax.experimental import pallas as pl
from jax.experimental.pallas import tpu as pltpu
from jax.experimental.pallas import tpu_sc as plsc

N = 10000
E = 320000
H = 128
T = 13
L = 6
EPS = 1e-5
BN_SCALE = 1.0 / np.sqrt(1.0 + EPS)

NC = 2
NS = 16
NW = NC * NS
IPC = 128
CPT = 80
EPT = IPC * CPT
EPAD = EPT * NW
NP = 10240
RPT = NP // NS

ROWB = 1024
GRID = NP // ROWB


def _mesh():
    return plsc.VectorSubcoreMesh(
        core_axis_name="c", subcore_axis_name="s", num_cores=NC, num_subcores=NS
    )


def _spmm_body(ms_hbm, row_hbm, col_hbm, zeros_hbm, out_hbm,
               cidx, rixr, rb0, rb1, acc,
               si0, si1, sg0, sg1, ss0, ss1):
    c = lax.axis_index("c")
    s = lax.axis_index("s")
    rows = (rb0, rb1)
    si = (si0, si1)
    sg = (sg0, sg1)
    ss = (ss0, ss1)

    pltpu.sync_copy(zeros_hbm.at[pl.ds(s * RPT, RPT)],
                    acc.at[pl.ds(s * RPT, RPT)])
    tb = (c * NS + s) * CPT
    pltpu.sync_copy(col_hbm.at[pl.ds(tb, CPT)], cidx)
    plsc.subcore_barrier()

    def fire_ridx(j, p):
        pltpu.async_copy(row_hbm.at[tb + j], rixr.at[p], si[p])

    def wait_ridx(j, p):
        pltpu.make_async_copy(row_hbm.at[tb + j], rixr.at[p], si[p]).wait()

    def fire_gather(p):
        pltpu.async_copy(ms_hbm.at[rixr.at[p]], rows[p], sg[p])

    def wait_gather(p):
        pltpu.make_async_copy(ms_hbm.at[rixr.at[p]], rows[p], sg[p]).wait()

    def fire_scatter(j, p):
        pltpu.async_copy(rows[p], acc.at[cidx.at[j]], ss[p], add=True)

    def wait_scatter(j, p):
        pltpu.make_async_copy(rows[p], acc.at[cidx.at[j]], ss[p]).wait()

    fire_ridx(0, 0)
    fire_ridx(1, 1)
    wait_ridx(0, 0)
    fire_gather(0)

    def half(j, p):
        q = 1 - p
        wait_gather(p)
        fire_scatter(j, p)

        @pl.when(j + 2 < CPT)
        def _():
            fire_ridx(j + 2, p)

        @pl.when(j > 0)
        def _():
            wait_scatter(j - 1, q)

        @pl.when(j + 1 < CPT)
        def _():
            wait_ridx(j + 1, q)
            fire_gather(q)

    def body(t, _):
        half(2 * t, 0)
        half(2 * t + 1, 1)
        return 0

    lax.fori_loop(0, CPT // 2, body, 0)
    wait_scatter(CPT - 1, 1)
    plsc.subcore_barrier()
    pltpu.sync_copy(acc.at[pl.ds(s * RPT, RPT)],
                    out_hbm.at[pl.ds(c * NP + s * RPT, RPT)])


def _spmm_call(ms, row2d, col2d, zeros128):
    k = pl.kernel(
        _spmm_body,
        out_type=jax.ShapeDtypeStruct((2 * NP, H), jnp.float32),
        mesh=_mesh(),
        scratch_types=[
            pltpu.VMEM((CPT, IPC), jnp.int32),
            pltpu.VMEM((2, IPC), jnp.int32),
            pltpu.VMEM((IPC, H), jnp.float32),
            pltpu.VMEM((IPC, H), jnp.float32),
            pltpu.VMEM_SHARED((NP, H), jnp.float32),
            pltpu.SemaphoreType.DMA,
            pltpu.SemaphoreType.DMA,
            pltpu.SemaphoreType.DMA,
            pltpu.SemaphoreType.DMA,
            pltpu.SemaphoreType.DMA,
            pltpu.SemaphoreType.DMA,
        ],
    )
    return k(ms, row2d, col2d, zeros128)


def _t1_body(x_ref, win_ref, bin_ref, wg0_ref, d0_ref, d1_ref,
             h0_ref, ms0_ref, dinv_ref):
    deg = 1.0 + d0_ref[:, 0:1] + d1_ref[:, 0:1]
    dv = lax.rsqrt(deg)
    h0 = jnp.dot(x_ref[...], win_ref[...],
                 preferred_element_type=jnp.float32) + bin_ref[...]
    h0_ref[...] = h0
    ms0_ref[...] = jnp.dot(h0, wg0_ref[...],
                           preferred_element_type=jnp.float32) * dv
    dinv_ref[...] = dv


def _t1_call(xp, W_in, b_in2, Wg0, degp):
    return pl.pallas_call(
        _t1_body,
        grid=(GRID,),
        in_specs=[
            pl.BlockSpec((ROWB, H), lambda i: (i, 0)),
            pl.BlockSpec((H, H), lambda i: (0, 0)),
            pl.BlockSpec((1, H), lambda i: (0, 0)),
            pl.BlockSpec((H, H), lambda i: (0, 0)),
            pl.BlockSpec((ROWB, H), lambda i: (i, 0)),
            pl.BlockSpec((ROWB, H), lambda i: (GRID + i, 0)),
        ],
        out_specs=[
            pl.BlockSpec((ROWB, H), lambda i: (i, 0)),
            pl.BlockSpec((ROWB, H), lambda i: (i, 0)),
            pl.BlockSpec((ROWB, 1), lambda i: (i, 0)),
        ],
        out_shape=[
            jax.ShapeDtypeStruct((NP, H), jnp.float32),
            jax.ShapeDtypeStruct((NP, H), jnp.float32),
            jax.ShapeDtypeStruct((NP, 1), jnp.float32),
        ],
    )(xp, W_in, b_in2, Wg0, degp, degp)


def _t2_body(p0_ref, p1_ref, ms_ref, h_ref, dinv_ref, bg_ref, gm_ref, bt_ref,
             wn_ref, hn_ref, msn_ref):
    dv = dinv_ref[...]
    agg = dv * (p0_ref[...] + p1_ref[...] + ms_ref[...]) + bg_ref[...]
    hb = agg * BN_SCALE * gm_ref[...] + bt_ref[...]
    hn = jnp.maximum(hb, 0.0) + h_ref[...]
    hn_ref[...] = hn
    msn_ref[...] = jnp.dot(hn, wn_ref[...],
                           preferred_element_type=jnp.float32) * dv


def _t2_call(P, ms, h, dinv, bg2, gm2, bt2, Wnext):
    return pl.pallas_call(
        _t2_body,
        grid=(GRID,),
        in_specs=[
            pl.BlockSpec((ROWB, H), lambda i: (i, 0)),
            pl.BlockSpec((ROWB, H), lambda i: (GRID + i, 0)),
            pl.BlockSpec((ROWB, H), lambda i: (i, 0)),
            pl.BlockSpec((ROWB, H), lambda i: (i, 0)),
            pl.BlockSpec((ROWB, 1), lambda i: (i, 0)),
            pl.BlockSpec((1, H), lambda i: (0, 0)),
            pl.BlockSpec((1, H), lambda i: (0, 0)),
            pl.BlockSpec((1, H), lambda i: (0, 0)),
            pl.BlockSpec((H, H), lambda i: (0, 0)),
        ],
        out_specs=[
            pl.BlockSpec((ROWB, H), lambda i: (i, 0)),
            pl.BlockSpec((ROWB, H), lambda i: (i, 0)),
        ],
        out_shape=[
            jax.ShapeDtypeStruct((NP, H), jnp.float32),
            jax.ShapeDtypeStruct((NP, H), jnp.float32),
        ],
    )(P, P, ms, h, dinv, bg2, gm2, bt2, Wnext)


def _t3_body(p0_ref, p1_ref, ms_ref, h_ref, dinv_ref, bg_ref, gm_ref, bt_ref,
             wo1_ref, bo1_ref, wo2_ref, bo2_ref, out_ref, acc):
    i = pl.program_id(0)
    dv = dinv_ref[...]
    agg = dv * (p0_ref[...] + p1_ref[...] + ms_ref[...]) + bg_ref[...]
    hb = agg * BN_SCALE * gm_ref[...] + bt_ref[...]
    hn = jnp.maximum(hb, 0.0) + h_ref[...]
    rid = i * ROWB + lax.broadcasted_iota(jnp.int32, (ROWB, 1), 0)
    hn = jnp.where(rid < N, hn, 0.0)
    bsum = jnp.sum(hn, axis=0, keepdims=True)

    @pl.when(i == 0)
    def _():
        acc[...] = bsum

    @pl.when(i > 0)
    def _():
        acc[...] = acc[...] + bsum

    @pl.when(i == pl.num_programs(0) - 1)
    def _():
        pooled = acc[...] * (1.0 / N)
        o = jnp.maximum(
            jnp.dot(pooled, wo1_ref[...],
                    preferred_element_type=jnp.float32) + bo1_ref[...], 0.0)
        out_ref[...] = jnp.dot(o, wo2_ref[...],
                               preferred_element_type=jnp.float32) + bo2_ref[...]


def _t3_call(P, ms, h, dinv, bg2, gm2, bt2, Wo1p, bo1p, Wo2p, bo2p):
    return pl.pallas_call(
        _t3_body,
        grid=(GRID,),
        in_specs=[
            pl.BlockSpec((ROWB, H), lambda i: (i, 0)),
            pl.BlockSpec((ROWB, H), lambda i: (GRID + i, 0)),
            pl.BlockSpec((ROWB, H), lambda i: (i, 0)),
            pl.BlockSpec((ROWB, H), lambda i: (i, 0)),
            pl.BlockSpec((ROWB, 1), lambda i: (i, 0)),
            pl.BlockSpec((1, H), lambda i: (0, 0)),
            pl.BlockSpec((1, H), lambda i: (0, 0)),
            pl.BlockSpec((1, H), lambda i: (0, 0)),
            pl.BlockSpec((H, H), lambda i: (0, 0)),
            pl.BlockSpec((1, H), lambda i: (0, 0)),
            pl.BlockSpec((H, H), lambda i: (0, 0)),
            pl.BlockSpec((1, H), lambda i: (0, 0)),
        ],
        out_specs=pl.BlockSpec((1, H), lambda i: (0, 0)),
        out_shape=jax.ShapeDtypeStruct((1, H), jnp.float32),
        scratch_shapes=[pltpu.VMEM((1, H), jnp.float32)],
    )(P, P, ms, h, dinv, bg2, gm2, bt2, Wo1p, bo1p, Wo2p, bo2p)


def kernel(x, edge_index, W_in, b_in, Wg, bg, gamma, beta, Wo1, bo1, Wo2, bo2):
    pad = EPAD - E
    row = jnp.concatenate([edge_index[0], jnp.zeros((pad,), jnp.int32)])
    col = jnp.concatenate([edge_index[1], jnp.full((pad,), N, jnp.int32)])
    row2d = row.reshape(EPAD // IPC, IPC)
    col2d = col.reshape(EPAD // IPC, IPC)
    zeros128 = jnp.zeros((NP, H), jnp.float32)
    ones128 = jnp.ones((NP, H), jnp.float32)
    xp = jnp.zeros((NP, H), jnp.float32).at[:N].set(x)

    b_in2 = b_in.reshape(1, H)
    bg2 = bg.reshape(L, 1, H)
    gm2 = gamma.reshape(L, 1, H)
    bt2 = beta.reshape(L, 1, H)
    Wo1p = jnp.zeros((H, H), jnp.float32).at[:, : H // 2].set(Wo1)
    bo1p = jnp.zeros((1, H), jnp.float32).at[:, : H // 2].set(bo1)
    Wo2p = jnp.zeros((H, H), jnp.float32).at[: H // 2, :T].set(Wo2)
    bo2p = jnp.zeros((1, H), jnp.float32).at[:, :T].set(bo2)

    degp = _spmm_call(ones128, col2d, col2d, zeros128)
    h, ms, dinv = _t1_call(xp, W_in, b_in2, Wg[0], degp)
    for l in range(L - 1):
        P = _spmm_call(ms, row2d, col2d, zeros128)
        h, ms = _t2_call(P, ms, h, dinv, bg2[l], gm2[l], bt2[l], Wg[l + 1])
    P = _spmm_call(ms, row2d, col2d, zeros128)
    out = _t3_call(P, ms, h, dinv, bg2[L - 1], gm2[L - 1], bt2[L - 1],
                   Wo1p, bo1p, Wo2p, bo2p)
    return out[:, :T]


# speedup vs baseline: 5.8239x; 1.0888x over previous
import jax
import jax.numpy as jnp
import numpy as np
from jax import lax
from jax.experimental import pallas as pl
from jax.experimental.pallas import tpu as pltpu
from jax.experimental.pallas import tpu_sc as plsc

N = 10000
E = 320000
H = 128
T = 13
L = 6
EPS = 1e-5
BN_SCALE = 1.0 / np.sqrt(1.0 + EPS)

NC = 2
NS = 16
NW = NC * NS
IPC = 128
CPT = 80
EPT = IPC * CPT
EPAD = EPT * NW
NP = 10240
RPT = NP // NS

ROWB = 1024
GRID = NP // ROWB


def _mesh():
    return plsc.VectorSubcoreMesh(
        core_axis_name="c", subcore_axis_name="s", num_cores=NC, num_subcores=NS
    )


DW = 128


def _deg_body(col_hbm, zeros_hbm, ones_hbm, out_hbm, cidx, ones_buf, acc,
              ss0, ss1):
    c = lax.axis_index("c")
    s = lax.axis_index("s")
    ss = (ss0, ss1)

    pltpu.sync_copy(zeros_hbm.at[pl.ds(s * RPT, RPT)],
                    acc.at[pl.ds(s * RPT, RPT)])
    pltpu.sync_copy(ones_hbm, ones_buf)
    tb = (c * NS + s) * CPT
    pltpu.sync_copy(col_hbm.at[pl.ds(tb, CPT)], cidx)
    plsc.subcore_barrier()

    def fire(j, p):
        pltpu.async_copy(ones_buf, acc.at[cidx.at[j]], ss[p], add=True)

    def wait(j, p):
        pltpu.make_async_copy(ones_buf, acc.at[cidx.at[j]], ss[p]).wait()

    fire(0, 0)
    fire(1, 1)

    def body(t, _):
        j = 2 * t
        wait(j, 0)

        @pl.when(j + 2 < CPT)
        def _():
            fire(j + 2, 0)

        wait(j + 1, 1)

        @pl.when(j + 3 < CPT)
        def _():
            fire(j + 3, 1)

        return 0

    lax.fori_loop(0, CPT // 2, body, 0)
    plsc.subcore_barrier()
    pltpu.sync_copy(acc.at[pl.ds(s * RPT, RPT)],
                    out_hbm.at[pl.ds(c * NP + s * RPT, RPT)])


def _deg_call(col2d, zeros16, ones16):
    k = pl.kernel(
        _deg_body,
        out_type=jax.ShapeDtypeStruct((2 * NP, DW), jnp.float32),
        mesh=_mesh(),
        scratch_types=[
            pltpu.VMEM((CPT, IPC), jnp.int32),
            pltpu.VMEM((IPC, DW), jnp.float32),
            pltpu.VMEM_SHARED((NP, DW), jnp.float32),
            pltpu.SemaphoreType.DMA,
            pltpu.SemaphoreType.DMA,
        ],
    )
    return k(col2d, zeros16, ones16)


def _spmm_body(ms_hbm, row_hbm, col_hbm, zeros_hbm, out_hbm,
               cidx, rixr, rb0, rb1, acc,
               si0, si1, sg0, sg1, ss0, ss1):
    c = lax.axis_index("c")
    s = lax.axis_index("s")
    rows = (rb0, rb1)
    si = (si0, si1)
    sg = (sg0, sg1)
    ss = (ss0, ss1)

    pltpu.sync_copy(zeros_hbm.at[pl.ds(s * RPT, RPT)],
                    acc.at[pl.ds(s * RPT, RPT)])
    tb = (c * NS + s) * CPT
    pltpu.sync_copy(col_hbm.at[pl.ds(tb, CPT)], cidx)
    plsc.subcore_barrier()

    def fire_ridx(j, p):
        pltpu.async_copy(row_hbm.at[tb + j], rixr.at[p], si[p])

    def wait_ridx(j, p):
        pltpu.make_async_copy(row_hbm.at[tb + j], rixr.at[p], si[p]).wait()

    def fire_gather(p):
        pltpu.async_copy(ms_hbm.at[rixr.at[p]], rows[p], sg[p])

    def wait_gather(p):
        pltpu.make_async_copy(ms_hbm.at[rixr.at[p]], rows[p], sg[p]).wait()

    def fire_scatter(j, p):
        pltpu.async_copy(rows[p], acc.at[cidx.at[j]], ss[p], add=True)

    def wait_scatter(j, p):
        pltpu.make_async_copy(rows[p], acc.at[cidx.at[j]], ss[p]).wait()

    fire_ridx(0, 0)
    fire_ridx(1, 1)
    wait_ridx(0, 0)
    fire_gather(0)

    def half(j, p):
        q = 1 - p
        wait_gather(p)
        fire_scatter(j, p)

        @pl.when(j + 2 < CPT)
        def _():
            fire_ridx(j + 2, p)

        @pl.when(j > 0)
        def _():
            wait_scatter(j - 1, q)

        @pl.when(j + 1 < CPT)
        def _():
            wait_ridx(j + 1, q)
            fire_gather(q)

    def body(t, _):
        half(2 * t, 0)
        half(2 * t + 1, 1)
        return 0

    lax.fori_loop(0, CPT // 2, body, 0)
    wait_scatter(CPT - 1, 1)
    plsc.subcore_barrier()
    pltpu.sync_copy(acc.at[pl.ds(s * RPT, RPT)],
                    out_hbm.at[pl.ds(c * NP + s * RPT, RPT)])


def _spmm_call(ms, row2d, col2d, zeros128):
    k = pl.kernel(
        _spmm_body,
        out_type=jax.ShapeDtypeStruct((2 * NP, H), jnp.float32),
        mesh=_mesh(),
        scratch_types=[
            pltpu.VMEM((CPT, IPC), jnp.int32),
            pltpu.VMEM((2, IPC), jnp.int32),
            pltpu.VMEM((IPC, H), jnp.float32),
            pltpu.VMEM((IPC, H), jnp.float32),
            pltpu.VMEM_SHARED((NP, H), jnp.float32),
            pltpu.SemaphoreType.DMA,
            pltpu.SemaphoreType.DMA,
            pltpu.SemaphoreType.DMA,
            pltpu.SemaphoreType.DMA,
            pltpu.SemaphoreType.DMA,
            pltpu.SemaphoreType.DMA,
        ],
    )
    return k(ms, row2d, col2d, zeros128)


def _t1_body(x_ref, win_ref, bin_ref, wg0_ref, d0_ref, d1_ref,
             h0_ref, ms0_ref, dinv_ref):
    deg = 1.0 + d0_ref[:, 0:1] + d1_ref[:, 0:1]
    dv = lax.rsqrt(deg)
    h0 = jnp.dot(x_ref[...], win_ref[...],
                 preferred_element_type=jnp.float32) + bin_ref[...]
    h0_ref[...] = h0
    ms0_ref[...] = jnp.dot(h0, wg0_ref[...],
                           preferred_element_type=jnp.float32) * dv
    dinv_ref[...] = dv


def _t1_call(xp, W_in, b_in2, Wg0, degp):
    return pl.pallas_call(
        _t1_body,
        grid=(GRID,),
        in_specs=[
            pl.BlockSpec((ROWB, H), lambda i: (i, 0)),
            pl.BlockSpec((H, H), lambda i: (0, 0)),
            pl.BlockSpec((1, H), lambda i: (0, 0)),
            pl.BlockSpec((H, H), lambda i: (0, 0)),
            pl.BlockSpec((ROWB, DW), lambda i: (i, 0)),
            pl.BlockSpec((ROWB, DW), lambda i: (GRID + i, 0)),
        ],
        out_specs=[
            pl.BlockSpec((ROWB, H), lambda i: (i, 0)),
            pl.BlockSpec((ROWB, H), lambda i: (i, 0)),
            pl.BlockSpec((ROWB, 1), lambda i: (i, 0)),
        ],
        out_shape=[
            jax.ShapeDtypeStruct((NP, H), jnp.float32),
            jax.ShapeDtypeStruct((NP, H), jnp.float32),
            jax.ShapeDtypeStruct((NP, 1), jnp.float32),
        ],
    )(xp, W_in, b_in2, Wg0, degp, degp)


def _t2_body(p0_ref, p1_ref, ms_ref, h_ref, dinv_ref, bg_ref, gm_ref, bt_ref,
             wn_ref, hn_ref, msn_ref):
    dv = dinv_ref[...]
    agg = dv * (p0_ref[...] + p1_ref[...] + ms_ref[...]) + bg_ref[...]
    hb = agg * BN_SCALE * gm_ref[...] + bt_ref[...]
    hn = jnp.maximum(hb, 0.0) + h_ref[...]
    hn_ref[...] = hn
    msn_ref[...] = jnp.dot(hn, wn_ref[...],
                           preferred_element_type=jnp.float32) * dv


def _t2_call(P, ms, h, dinv, bg2, gm2, bt2, Wnext):
    return pl.pallas_call(
        _t2_body,
        grid=(GRID,),
        in_specs=[
            pl.BlockSpec((ROWB, H), lambda i: (i, 0)),
            pl.BlockSpec((ROWB, H), lambda i: (GRID + i, 0)),
            pl.BlockSpec((ROWB, H), lambda i: (i, 0)),
            pl.BlockSpec((ROWB, H), lambda i: (i, 0)),
            pl.BlockSpec((ROWB, 1), lambda i: (i, 0)),
            pl.BlockSpec((1, H), lambda i: (0, 0)),
            pl.BlockSpec((1, H), lambda i: (0, 0)),
            pl.BlockSpec((1, H), lambda i: (0, 0)),
            pl.BlockSpec((H, H), lambda i: (0, 0)),
        ],
        out_specs=[
            pl.BlockSpec((ROWB, H), lambda i: (i, 0)),
            pl.BlockSpec((ROWB, H), lambda i: (i, 0)),
        ],
        out_shape=[
            jax.ShapeDtypeStruct((NP, H), jnp.float32),
            jax.ShapeDtypeStruct((NP, H), jnp.float32),
        ],
    )(P, P, ms, h, dinv, bg2, gm2, bt2, Wnext)


def _t3_body(p0_ref, p1_ref, ms_ref, h_ref, dinv_ref, bg_ref, gm_ref, bt_ref,
             wo1_ref, bo1_ref, wo2_ref, bo2_ref, out_ref, acc):
    i = pl.program_id(0)
    dv = dinv_ref[...]
    agg = dv * (p0_ref[...] + p1_ref[...] + ms_ref[...]) + bg_ref[...]
    hb = agg * BN_SCALE * gm_ref[...] + bt_ref[...]
    hn = jnp.maximum(hb, 0.0) + h_ref[...]
    rid = i * ROWB + lax.broadcasted_iota(jnp.int32, (ROWB, 1), 0)
    hn = jnp.where(rid < N, hn, 0.0)
    bsum = jnp.sum(hn, axis=0, keepdims=True)

    @pl.when(i == 0)
    def _():
        acc[...] = bsum

    @pl.when(i > 0)
    def _():
        acc[...] = acc[...] + bsum

    @pl.when(i == pl.num_programs(0) - 1)
    def _():
        pooled = acc[...] * (1.0 / N)
        o = jnp.maximum(
            jnp.dot(pooled, wo1_ref[...],
                    preferred_element_type=jnp.float32) + bo1_ref[...], 0.0)
        out_ref[...] = jnp.dot(o, wo2_ref[...],
                               preferred_element_type=jnp.float32) + bo2_ref[...]


def _t3_call(P, ms, h, dinv, bg2, gm2, bt2, Wo1p, bo1p, Wo2p, bo2p):
    return pl.pallas_call(
        _t3_body,
        grid=(GRID,),
        in_specs=[
            pl.BlockSpec((ROWB, H), lambda i: (i, 0)),
            pl.BlockSpec((ROWB, H), lambda i: (GRID + i, 0)),
            pl.BlockSpec((ROWB, H), lambda i: (i, 0)),
            pl.BlockSpec((ROWB, H), lambda i: (i, 0)),
            pl.BlockSpec((ROWB, 1), lambda i: (i, 0)),
            pl.BlockSpec((1, H), lambda i: (0, 0)),
            pl.BlockSpec((1, H), lambda i: (0, 0)),
            pl.BlockSpec((1, H), lambda i: (0, 0)),
            pl.BlockSpec((H, H), lambda i: (0, 0)),
            pl.BlockSpec((1, H), lambda i: (0, 0)),
            pl.BlockSpec((H, H), lambda i: (0, 0)),
            pl.BlockSpec((1, H), lambda i: (0, 0)),
        ],
        out_specs=pl.BlockSpec((1, H), lambda i: (0, 0)),
        out_shape=jax.ShapeDtypeStruct((1, H), jnp.float32),
        scratch_shapes=[pltpu.VMEM((1, H), jnp.float32)],
    )(P, P, ms, h, dinv, bg2, gm2, bt2, Wo1p, bo1p, Wo2p, bo2p)


def kernel(x, edge_index, W_in, b_in, Wg, bg, gamma, beta, Wo1, bo1, Wo2, bo2):
    pad = EPAD - E
    row = jnp.concatenate([edge_index[0], jnp.zeros((pad,), jnp.int32)])
    col = jnp.concatenate([edge_index[1], jnp.full((pad,), N, jnp.int32)])
    row2d = row.reshape(EPAD // IPC, IPC)
    col2d = col.reshape(EPAD // IPC, IPC)
    zeros128 = jnp.zeros((NP, H), jnp.float32)
    ones16 = jnp.ones((IPC, DW), jnp.float32)
    xp = jnp.zeros((NP, H), jnp.float32).at[:N].set(x)

    b_in2 = b_in.reshape(1, H)
    bg2 = bg.reshape(L, 1, H)
    gm2 = gamma.reshape(L, 1, H)
    bt2 = beta.reshape(L, 1, H)
    Wo1p = jnp.zeros((H, H), jnp.float32).at[:, : H // 2].set(Wo1)
    bo1p = jnp.zeros((1, H), jnp.float32).at[:, : H // 2].set(bo1)
    Wo2p = jnp.zeros((H, H), jnp.float32).at[: H // 2, :T].set(Wo2)
    bo2p = jnp.zeros((1, H), jnp.float32).at[:, :T].set(bo2)

    degp = _deg_call(col2d, zeros128, ones16)
    h, ms, dinv = _t1_call(xp, W_in, b_in2, Wg[0], degp)
    for l in range(L - 1):
        P = _spmm_call(ms, row2d, col2d, zeros128)
        h, ms = _t2_call(P, ms, h, dinv, bg2[l], gm2[l], bt2[l], Wg[l + 1])
    P = _spmm_call(ms, row2d, col2d, zeros128)
    out = _t3_call(P, ms, h, dinv, bg2[L - 1], gm2[L - 1], bt2[L - 1],
                   Wo1p, bo1p, Wo2p, bo2p)
    return out[:, :T]


# speedup vs baseline: 18.8059x; 3.2291x over previous
import jax
import jax.numpy as jnp
import numpy as np
from jax import lax
from jax.experimental import pallas as pl
from jax.experimental.pallas import tpu as pltpu
from jax.experimental.pallas import tpu_sc as plsc

N = 10000
E = 320000
H = 128
T = 13
L = 6
EPS = 1e-5
BN_SCALE = 1.0 / np.sqrt(1.0 + EPS)

NC = 2
NS = 16
NW = NC * NS
IPC = 128
CPT = 80
EPT = IPC * CPT
EPAD = EPT * NW
NP = 10240
RPT = NP // NS

ROWB = 1024
GRID = NP // ROWB


def _mesh():
    return plsc.VectorSubcoreMesh(
        core_axis_name="c", subcore_axis_name="s", num_cores=NC, num_subcores=NS
    )


DW = 128


def _deg_body(col_hbm, zeros_hbm, ones_hbm, out_hbm, cidx, ones_buf, acc,
              ss0, ss1):
    c = lax.axis_index("c")
    s = lax.axis_index("s")
    ss = (ss0, ss1)

    pltpu.sync_copy(zeros_hbm.at[pl.ds(s * RPT, RPT)],
                    acc.at[pl.ds(s * RPT, RPT)])
    pltpu.sync_copy(ones_hbm, ones_buf)
    tb = (c * NS + s) * CPT
    pltpu.sync_copy(col_hbm.at[pl.ds(tb, CPT)], cidx)
    plsc.subcore_barrier()

    def fire(j, p):
        pltpu.async_copy(ones_buf, acc.at[cidx.at[j]], ss[p], add=True)

    def wait(j, p):
        pltpu.make_async_copy(ones_buf, acc.at[cidx.at[j]], ss[p]).wait()

    fire(0, 0)
    fire(1, 1)

    def body(t, _):
        j = 2 * t
        wait(j, 0)

        @pl.when(j + 2 < CPT)
        def _():
            fire(j + 2, 0)

        wait(j + 1, 1)

        @pl.when(j + 3 < CPT)
        def _():
            fire(j + 3, 1)

        return 0

    lax.fori_loop(0, CPT // 2, body, 0)
    plsc.subcore_barrier()
    pltpu.sync_copy(acc.at[pl.ds(s * RPT, RPT)],
                    out_hbm.at[pl.ds(c * NP + s * RPT, RPT)])


def _deg_call(col2d, zeros16, ones16):
    k = pl.kernel(
        _deg_body,
        out_type=jax.ShapeDtypeStruct((2 * NP, DW), jnp.float32),
        mesh=_mesh(),
        scratch_types=[
            pltpu.VMEM((CPT, IPC), jnp.int32),
            pltpu.VMEM((IPC, DW), jnp.float32),
            pltpu.VMEM_SHARED((NP, DW), jnp.float32),
            pltpu.SemaphoreType.DMA,
            pltpu.SemaphoreType.DMA,
        ],
    )
    return k(col2d, zeros16, ones16)


def _spmm_body(ms_hbm, row_hbm, col_hbm, zeros_hbm, out_hbm,
               cidx, rixr, rb0, rb1, acc,
               si0, si1, sg0, sg1, ss0, ss1):
    c = lax.axis_index("c")
    s = lax.axis_index("s")
    rows = (rb0, rb1)
    si = (si0, si1)
    sg = (sg0, sg1)
    ss = (ss0, ss1)

    pltpu.sync_copy(zeros_hbm.at[pl.ds(s * RPT, RPT)],
                    acc.at[pl.ds(s * RPT, RPT)])
    tb = (c * NS + s) * CPT
    pltpu.sync_copy(col_hbm.at[pl.ds(tb, CPT)], cidx)
    plsc.subcore_barrier()

    def fire_ridx(j, p):
        pltpu.async_copy(row_hbm.at[tb + j], rixr.at[p], si[p])

    def wait_ridx(j, p):
        pltpu.make_async_copy(row_hbm.at[tb + j], rixr.at[p], si[p]).wait()

    def fire_gather(p):
        pltpu.async_copy(ms_hbm.at[rixr.at[p]], rows[p], sg[p])

    def wait_gather(p):
        pltpu.make_async_copy(ms_hbm.at[rixr.at[p]], rows[p], sg[p]).wait()

    def fire_scatter(j, p):
        pltpu.async_copy(rows[p], acc.at[cidx.at[j]], ss[p], add=True)

    def wait_scatter(j, p):
        pltpu.make_async_copy(rows[p], acc.at[cidx.at[j]], ss[p]).wait()

    fire_ridx(0, 0)
    fire_ridx(1, 1)
    wait_ridx(0, 0)
    fire_gather(0)

    def half(j, p):
        q = 1 - p
        wait_gather(p)
        fire_scatter(j, p)

        @pl.when(j + 2 < CPT)
        def _():
            fire_ridx(j + 2, p)

        @pl.when(j > 0)
        def _():
            wait_scatter(j - 1, q)

        @pl.when(j + 1 < CPT)
        def _():
            wait_ridx(j + 1, q)
            fire_gather(q)

    def body(t, _):
        half(2 * t, 0)
        half(2 * t + 1, 1)
        return 0

    lax.fori_loop(0, CPT // 2, body, 0)
    wait_scatter(CPT - 1, 1)
    plsc.subcore_barrier()
    pltpu.sync_copy(acc.at[pl.ds(s * RPT, RPT)],
                    out_hbm.at[pl.ds(c * NP + s * RPT, RPT)])


def _spmm_call(ms, row2d, col2d, zeros128):
    k = pl.kernel(
        _spmm_body,
        out_type=jax.ShapeDtypeStruct((2 * NP, H), jnp.float32),
        mesh=_mesh(),
        scratch_types=[
            pltpu.VMEM((CPT, IPC), jnp.int32),
            pltpu.VMEM((2, IPC), jnp.int32),
            pltpu.VMEM((IPC, H), jnp.float32),
            pltpu.VMEM((IPC, H), jnp.float32),
            pltpu.VMEM_SHARED((NP, H), jnp.float32),
            pltpu.SemaphoreType.DMA,
            pltpu.SemaphoreType.DMA,
            pltpu.SemaphoreType.DMA,
            pltpu.SemaphoreType.DMA,
            pltpu.SemaphoreType.DMA,
            pltpu.SemaphoreType.DMA,
        ],
    )
    return k(ms, row2d, col2d, zeros128)


def _t1_body(x_ref, win_ref, bin_ref, wg0_ref, d0_ref, d1_ref,
             h0_ref, ms0_ref, dinv_ref):
    deg = 1.0 + d0_ref[:, 0:1] + d1_ref[:, 0:1]
    dv = lax.rsqrt(deg)
    h0 = jnp.dot(x_ref[...], win_ref[...],
                 preferred_element_type=jnp.float32) + bin_ref[...]
    h0_ref[...] = h0
    ms0_ref[...] = jnp.dot(h0, wg0_ref[...],
                           preferred_element_type=jnp.float32) * dv
    dinv_ref[...] = dv


def _t1_call(xp, W_in, b_in2, Wg0, degp):
    return pl.pallas_call(
        _t1_body,
        grid=(GRID,),
        in_specs=[
            pl.BlockSpec((ROWB, H), lambda i: (i, 0)),
            pl.BlockSpec((H, H), lambda i: (0, 0)),
            pl.BlockSpec((1, H), lambda i: (0, 0)),
            pl.BlockSpec((H, H), lambda i: (0, 0)),
            pl.BlockSpec((ROWB, DW), lambda i: (i, 0)),
            pl.BlockSpec((ROWB, DW), lambda i: (GRID + i, 0)),
        ],
        out_specs=[
            pl.BlockSpec((ROWB, H), lambda i: (i, 0)),
            pl.BlockSpec((ROWB, H), lambda i: (i, 0)),
            pl.BlockSpec((ROWB, 1), lambda i: (i, 0)),
        ],
        out_shape=[
            jax.ShapeDtypeStruct((NP, H), jnp.float32),
            jax.ShapeDtypeStruct((NP, H), jnp.float32),
            jax.ShapeDtypeStruct((NP, 1), jnp.float32),
        ],
    )(xp, W_in, b_in2, Wg0, degp, degp)


def _t2_body(p0_ref, p1_ref, ms_ref, h_ref, dinv_ref, bg_ref, gm_ref, bt_ref,
             wn_ref, hn_ref, msn_ref):
    dv = dinv_ref[...]
    agg = dv * (p0_ref[...] + p1_ref[...] + ms_ref[...]) + bg_ref[...]
    hb = agg * BN_SCALE * gm_ref[...] + bt_ref[...]
    hn = jnp.maximum(hb, 0.0) + h_ref[...]
    hn_ref[...] = hn
    msn_ref[...] = jnp.dot(hn, wn_ref[...],
                           preferred_element_type=jnp.float32) * dv


def _t2_call(P, ms, h, dinv, bg2, gm2, bt2, Wnext):
    return pl.pallas_call(
        _t2_body,
        grid=(GRID,),
        in_specs=[
            pl.BlockSpec((ROWB, H), lambda i: (i, 0)),
            pl.BlockSpec((ROWB, H), lambda i: (GRID + i, 0)),
            pl.BlockSpec((ROWB, H), lambda i: (i, 0)),
            pl.BlockSpec((ROWB, H), lambda i: (i, 0)),
            pl.BlockSpec((ROWB, 1), lambda i: (i, 0)),
            pl.BlockSpec((1, H), lambda i: (0, 0)),
            pl.BlockSpec((1, H), lambda i: (0, 0)),
            pl.BlockSpec((1, H), lambda i: (0, 0)),
            pl.BlockSpec((H, H), lambda i: (0, 0)),
        ],
        out_specs=[
            pl.BlockSpec((ROWB, H), lambda i: (i, 0)),
            pl.BlockSpec((ROWB, H), lambda i: (i, 0)),
        ],
        out_shape=[
            jax.ShapeDtypeStruct((NP, H), jnp.float32),
            jax.ShapeDtypeStruct((NP, H), jnp.float32),
        ],
    )(P, P, ms, h, dinv, bg2, gm2, bt2, Wnext)


def _t3_body(p0_ref, p1_ref, ms_ref, h_ref, dinv_ref, bg_ref, gm_ref, bt_ref,
             wo1_ref, bo1_ref, wo2_ref, bo2_ref, out_ref, acc):
    i = pl.program_id(0)
    dv = dinv_ref[...]
    agg = dv * (p0_ref[...] + p1_ref[...] + ms_ref[...]) + bg_ref[...]
    hb = agg * BN_SCALE * gm_ref[...] + bt_ref[...]
    hn = jnp.maximum(hb, 0.0) + h_ref[...]
    rid = i * ROWB + lax.broadcasted_iota(jnp.int32, (ROWB, 1), 0)
    hn = jnp.where(rid < N, hn, 0.0)
    bsum = jnp.sum(hn, axis=0, keepdims=True)

    @pl.when(i == 0)
    def _():
        acc[...] = bsum

    @pl.when(i > 0)
    def _():
        acc[...] = acc[...] + bsum

    @pl.when(i == pl.num_programs(0) - 1)
    def _():
        pooled = acc[...] * (1.0 / N)
        o = jnp.maximum(
            jnp.dot(pooled, wo1_ref[...],
                    preferred_element_type=jnp.float32) + bo1_ref[...], 0.0)
        out_ref[...] = jnp.dot(o, wo2_ref[...],
                               preferred_element_type=jnp.float32) + bo2_ref[...]


def _t3_call(P, ms, h, dinv, bg2, gm2, bt2, Wo1p, bo1p, Wo2p, bo2p):
    return pl.pallas_call(
        _t3_body,
        grid=(GRID,),
        in_specs=[
            pl.BlockSpec((ROWB, H), lambda i: (i, 0)),
            pl.BlockSpec((ROWB, H), lambda i: (GRID + i, 0)),
            pl.BlockSpec((ROWB, H), lambda i: (i, 0)),
            pl.BlockSpec((ROWB, H), lambda i: (i, 0)),
            pl.BlockSpec((ROWB, 1), lambda i: (i, 0)),
            pl.BlockSpec((1, H), lambda i: (0, 0)),
            pl.BlockSpec((1, H), lambda i: (0, 0)),
            pl.BlockSpec((1, H), lambda i: (0, 0)),
            pl.BlockSpec((H, H), lambda i: (0, 0)),
            pl.BlockSpec((1, H), lambda i: (0, 0)),
            pl.BlockSpec((H, H), lambda i: (0, 0)),
            pl.BlockSpec((1, H), lambda i: (0, 0)),
        ],
        out_specs=pl.BlockSpec((1, H), lambda i: (0, 0)),
        out_shape=jax.ShapeDtypeStruct((1, H), jnp.float32),
        scratch_shapes=[pltpu.VMEM((1, H), jnp.float32)],
    )(P, P, ms, h, dinv, bg2, gm2, bt2, Wo1p, bo1p, Wo2p, bo2p)


def kernel(x, edge_index, W_in, b_in, Wg, bg, gamma, beta, Wo1, bo1, Wo2, bo2):
    pad = EPAD - E
    pidx = jnp.arange(pad, dtype=jnp.int32)
    row = jnp.concatenate([edge_index[0], pidx % np.int32(N)])
    col = jnp.concatenate([edge_index[1], N + pidx % np.int32(NP - N)])
    row2d = row.reshape(EPAD // IPC, IPC)
    col2d = col.reshape(EPAD // IPC, IPC)
    zeros128 = jnp.zeros((NP, H), jnp.float32)
    ones16 = jnp.ones((IPC, DW), jnp.float32)
    xp = jnp.zeros((NP, H), jnp.float32).at[:N].set(x)

    b_in2 = b_in.reshape(1, H)
    bg2 = bg.reshape(L, 1, H)
    gm2 = gamma.reshape(L, 1, H)
    bt2 = beta.reshape(L, 1, H)
    Wo1p = jnp.zeros((H, H), jnp.float32).at[:, : H // 2].set(Wo1)
    bo1p = jnp.zeros((1, H), jnp.float32).at[:, : H // 2].set(bo1)
    Wo2p = jnp.zeros((H, H), jnp.float32).at[: H // 2, :T].set(Wo2)
    bo2p = jnp.zeros((1, H), jnp.float32).at[:, :T].set(bo2)

    degp = _deg_call(col2d, zeros128, ones16)
    h, ms, dinv = _t1_call(xp, W_in, b_in2, Wg[0], degp)
    for l in range(L - 1):
        P = _spmm_call(ms, row2d, col2d, zeros128)
        h, ms = _t2_call(P, ms, h, dinv, bg2[l], gm2[l], bt2[l], Wg[l + 1])
    P = _spmm_call(ms, row2d, col2d, zeros128)
    out = _t3_call(P, ms, h, dinv, bg2[L - 1], gm2[L - 1], bt2[L - 1],
                   Wo1p, bo1p, Wo2p, bo2p)
    return out[:, :T]


# speedup vs baseline: 18.9398x; 1.0071x over previous
import jax
import jax.numpy as jnp
import numpy as np
from jax import lax
from jax.experimental import pallas as pl
from jax.experimental.pallas import tpu as pltpu
from jax.experimental.pallas import tpu_sc as plsc

N = 10000
E = 320000
H = 128
T = 13
L = 6
EPS = 1e-5
BN_SCALE = 1.0 / np.sqrt(1.0 + EPS)

NC = 2
NS = 16
NW = NC * NS
IPC = 128
CPT = 80
EPT = IPC * CPT
EPAD = EPT * NW
NP = 10240
RPT = NP // NS

ROWB = 1024
GRID = NP // ROWB


def _mesh():
    return plsc.VectorSubcoreMesh(
        core_axis_name="c", subcore_axis_name="s", num_cores=NC, num_subcores=NS
    )


DW = 128


def _deg_body(col_hbm, zeros_hbm, ones_hbm, out_hbm, cidx, ones_buf, acc,
              ss0, ss1):
    c = lax.axis_index("c")
    s = lax.axis_index("s")
    ss = (ss0, ss1)

    pltpu.sync_copy(zeros_hbm.at[pl.ds(s * RPT, RPT)],
                    acc.at[pl.ds(s * RPT, RPT)])
    pltpu.sync_copy(ones_hbm, ones_buf)
    tb = (c * NS + s) * CPT
    pltpu.sync_copy(col_hbm.at[pl.ds(tb, CPT)], cidx)
    plsc.subcore_barrier()

    def fire(j, p):
        pltpu.async_copy(ones_buf, acc.at[cidx.at[j]], ss[p], add=True)

    def wait(j, p):
        pltpu.make_async_copy(ones_buf, acc.at[cidx.at[j]], ss[p]).wait()

    fire(0, 0)
    fire(1, 1)

    def body(t, _):
        j = 2 * t
        wait(j, 0)

        @pl.when(j + 2 < CPT)
        def _():
            fire(j + 2, 0)

        wait(j + 1, 1)

        @pl.when(j + 3 < CPT)
        def _():
            fire(j + 3, 1)

        return 0

    lax.fori_loop(0, CPT // 2, body, 0)
    plsc.subcore_barrier()
    pltpu.sync_copy(acc.at[pl.ds(s * RPT, RPT)],
                    out_hbm.at[pl.ds(c * NP + s * RPT, RPT)])


def _deg_call(col2d, zeros16, ones16):
    k = pl.kernel(
        _deg_body,
        out_type=jax.ShapeDtypeStruct((2 * NP, DW), jnp.float32),
        mesh=_mesh(),
        scratch_types=[
            pltpu.VMEM((CPT, IPC), jnp.int32),
            pltpu.VMEM((IPC, DW), jnp.float32),
            pltpu.VMEM_SHARED((NP, DW), jnp.float32),
            pltpu.SemaphoreType.DMA,
            pltpu.SemaphoreType.DMA,
        ],
    )
    return k(col2d, zeros16, ones16)


def _spmm_body(ms_hbm, row_hbm, col_hbm, zeros_hbm, out_hbm,
               cidx, rixr, rb0, rb1, acc,
               si0, si1, sg0, sg1, ss0, ss1):
    c = lax.axis_index("c")
    s = lax.axis_index("s")
    rows = (rb0, rb1)
    si = (si0, si1)
    sg = (sg0, sg1)
    ss = (ss0, ss1)

    @pl.when(c == 0)
    def _():
        pltpu.sync_copy(ms_hbm.at[pl.ds(s * RPT, RPT)],
                        acc.at[pl.ds(s * RPT, RPT)])

    @pl.when(c == 1)
    def _():
        pltpu.sync_copy(zeros_hbm.at[pl.ds(s * RPT, RPT)],
                        acc.at[pl.ds(s * RPT, RPT)])

    tb = (c * NS + s) * CPT
    pltpu.sync_copy(col_hbm.at[pl.ds(tb, CPT)], cidx)
    plsc.subcore_barrier()

    def fire_ridx(j, p):
        pltpu.async_copy(row_hbm.at[tb + j], rixr.at[p], si[p])

    def wait_ridx(j, p):
        pltpu.make_async_copy(row_hbm.at[tb + j], rixr.at[p], si[p]).wait()

    def fire_gather(p):
        pltpu.async_copy(ms_hbm.at[rixr.at[p]], rows[p], sg[p])

    def wait_gather(p):
        pltpu.make_async_copy(ms_hbm.at[rixr.at[p]], rows[p], sg[p]).wait()

    def fire_scatter(j, p):
        pltpu.async_copy(rows[p], acc.at[cidx.at[j]], ss[p], add=True)

    def wait_scatter(j, p):
        pltpu.make_async_copy(rows[p], acc.at[cidx.at[j]], ss[p]).wait()

    fire_ridx(0, 0)
    fire_ridx(1, 1)
    wait_ridx(0, 0)
    fire_gather(0)

    def half(j, p):
        q = 1 - p
        wait_gather(p)
        fire_scatter(j, p)

        @pl.when(j + 2 < CPT)
        def _():
            fire_ridx(j + 2, p)

        @pl.when(j > 0)
        def _():
            wait_scatter(j - 1, q)

        @pl.when(j + 1 < CPT)
        def _():
            wait_ridx(j + 1, q)
            fire_gather(q)

    def body(t, _):
        half(2 * t, 0)
        half(2 * t + 1, 1)
        return 0

    lax.fori_loop(0, CPT // 2, body, 0)
    wait_scatter(CPT - 1, 1)
    plsc.subcore_barrier()
    pltpu.sync_copy(acc.at[pl.ds(s * RPT, RPT)],
                    out_hbm.at[pl.ds(c * NP + s * RPT, RPT)])


def _spmm_call(ms, row2d, col2d, zeros128):
    k = pl.kernel(
        _spmm_body,
        out_type=jax.ShapeDtypeStruct((2 * NP, H), jnp.float32),
        mesh=_mesh(),
        scratch_types=[
            pltpu.VMEM((CPT, IPC), jnp.int32),
            pltpu.VMEM((2, IPC), jnp.int32),
            pltpu.VMEM((IPC, H), jnp.float32),
            pltpu.VMEM((IPC, H), jnp.float32),
            pltpu.VMEM_SHARED((NP, H), jnp.float32),
            pltpu.SemaphoreType.DMA,
            pltpu.SemaphoreType.DMA,
            pltpu.SemaphoreType.DMA,
            pltpu.SemaphoreType.DMA,
            pltpu.SemaphoreType.DMA,
            pltpu.SemaphoreType.DMA,
        ],
    )
    return k(ms, row2d, col2d, zeros128)


def _t1_body(x_ref, win_ref, bin_ref, wg0_ref, d0_ref, d1_ref,
             h0_ref, ms0_ref, dinv_ref):
    deg = 1.0 + d0_ref[:, 0:1] + d1_ref[:, 0:1]
    dv = lax.rsqrt(deg)
    h0 = jnp.dot(x_ref[...], win_ref[...],
                 preferred_element_type=jnp.float32) + bin_ref[...]
    h0_ref[...] = h0
    ms0_ref[...] = jnp.dot(h0, wg0_ref[...],
                           preferred_element_type=jnp.float32) * dv
    dinv_ref[...] = dv


def _t1_call(xp, W_in, b_in2, Wg0, degp):
    return pl.pallas_call(
        _t1_body,
        grid=(GRID,),
        in_specs=[
            pl.BlockSpec((ROWB, H), lambda i: (i, 0)),
            pl.BlockSpec((H, H), lambda i: (0, 0)),
            pl.BlockSpec((1, H), lambda i: (0, 0)),
            pl.BlockSpec((H, H), lambda i: (0, 0)),
            pl.BlockSpec((ROWB, DW), lambda i: (i, 0)),
            pl.BlockSpec((ROWB, DW), lambda i: (GRID + i, 0)),
        ],
        out_specs=[
            pl.BlockSpec((ROWB, H), lambda i: (i, 0)),
            pl.BlockSpec((ROWB, H), lambda i: (i, 0)),
            pl.BlockSpec((ROWB, 1), lambda i: (i, 0)),
        ],
        out_shape=[
            jax.ShapeDtypeStruct((NP, H), jnp.float32),
            jax.ShapeDtypeStruct((NP, H), jnp.float32),
            jax.ShapeDtypeStruct((NP, 1), jnp.float32),
        ],
    )(xp, W_in, b_in2, Wg0, degp, degp)


def _t2_body(p0_ref, p1_ref, h_ref, dinv_ref, bg_ref, gm_ref, bt_ref,
             wn_ref, hn_ref, msn_ref):
    dv = dinv_ref[...]
    agg = dv * (p0_ref[...] + p1_ref[...]) + bg_ref[...]
    hb = agg * BN_SCALE * gm_ref[...] + bt_ref[...]
    hn = jnp.maximum(hb, 0.0) + h_ref[...]
    hn_ref[...] = hn
    msn_ref[...] = jnp.dot(hn, wn_ref[...],
                           preferred_element_type=jnp.float32) * dv


def _t2_call(P, h, dinv, bg2, gm2, bt2, Wnext):
    return pl.pallas_call(
        _t2_body,
        grid=(GRID,),
        in_specs=[
            pl.BlockSpec((ROWB, H), lambda i: (i, 0)),
            pl.BlockSpec((ROWB, H), lambda i: (GRID + i, 0)),
            pl.BlockSpec((ROWB, H), lambda i: (i, 0)),
            pl.BlockSpec((ROWB, 1), lambda i: (i, 0)),
            pl.BlockSpec((1, H), lambda i: (0, 0)),
            pl.BlockSpec((1, H), lambda i: (0, 0)),
            pl.BlockSpec((1, H), lambda i: (0, 0)),
            pl.BlockSpec((H, H), lambda i: (0, 0)),
        ],
        out_specs=[
            pl.BlockSpec((ROWB, H), lambda i: (i, 0)),
            pl.BlockSpec((ROWB, H), lambda i: (i, 0)),
        ],
        out_shape=[
            jax.ShapeDtypeStruct((NP, H), jnp.float32),
            jax.ShapeDtypeStruct((NP, H), jnp.float32),
        ],
    )(P, P, h, dinv, bg2, gm2, bt2, Wnext)


def _t3_body(p0_ref, p1_ref, h_ref, dinv_ref, bg_ref, gm_ref, bt_ref,
             wo1_ref, bo1_ref, wo2_ref, bo2_ref, out_ref, acc):
    i = pl.program_id(0)
    dv = dinv_ref[...]
    agg = dv * (p0_ref[...] + p1_ref[...]) + bg_ref[...]
    hb = agg * BN_SCALE * gm_ref[...] + bt_ref[...]
    hn = jnp.maximum(hb, 0.0) + h_ref[...]
    rid = i * ROWB + lax.broadcasted_iota(jnp.int32, (ROWB, 1), 0)
    hn = jnp.where(rid < N, hn, 0.0)
    bsum = jnp.sum(hn, axis=0, keepdims=True)

    @pl.when(i == 0)
    def _():
        acc[...] = bsum

    @pl.when(i > 0)
    def _():
        acc[...] = acc[...] + bsum

    @pl.when(i == pl.num_programs(0) - 1)
    def _():
        pooled = acc[...] * (1.0 / N)
        o = jnp.maximum(
            jnp.dot(pooled, wo1_ref[...],
                    preferred_element_type=jnp.float32) + bo1_ref[...], 0.0)
        out_ref[...] = jnp.dot(o, wo2_ref[...],
                               preferred_element_type=jnp.float32) + bo2_ref[...]


def _t3_call(P, h, dinv, bg2, gm2, bt2, Wo1p, bo1p, Wo2p, bo2p):
    return pl.pallas_call(
        _t3_body,
        grid=(GRID,),
        in_specs=[
            pl.BlockSpec((ROWB, H), lambda i: (i, 0)),
            pl.BlockSpec((ROWB, H), lambda i: (GRID + i, 0)),
            pl.BlockSpec((ROWB, H), lambda i: (i, 0)),
            pl.BlockSpec((ROWB, 1), lambda i: (i, 0)),
            pl.BlockSpec((1, H), lambda i: (0, 0)),
            pl.BlockSpec((1, H), lambda i: (0, 0)),
            pl.BlockSpec((1, H), lambda i: (0, 0)),
            pl.BlockSpec((H, H), lambda i: (0, 0)),
            pl.BlockSpec((1, H), lambda i: (0, 0)),
            pl.BlockSpec((H, H), lambda i: (0, 0)),
            pl.BlockSpec((1, H), lambda i: (0, 0)),
        ],
        out_specs=pl.BlockSpec((1, H), lambda i: (0, 0)),
        out_shape=jax.ShapeDtypeStruct((1, H), jnp.float32),
        scratch_shapes=[pltpu.VMEM((1, H), jnp.float32)],
    )(P, P, h, dinv, bg2, gm2, bt2, Wo1p, bo1p, Wo2p, bo2p)


def kernel(x, edge_index, W_in, b_in, Wg, bg, gamma, beta, Wo1, bo1, Wo2, bo2):
    pad = EPAD - E
    pidx = jnp.arange(pad, dtype=jnp.int32)
    row = jnp.concatenate([edge_index[0], pidx % np.int32(N)])
    col = jnp.concatenate([edge_index[1], N + pidx % np.int32(NP - N)])
    row2d = row.reshape(EPAD // IPC, IPC)
    col2d = col.reshape(EPAD // IPC, IPC)
    zeros128 = jnp.zeros((NP, H), jnp.float32)
    ones16 = jnp.ones((IPC, DW), jnp.float32)
    xp = jnp.zeros((NP, H), jnp.float32).at[:N].set(x)

    b_in2 = b_in.reshape(1, H)
    bg2 = bg.reshape(L, 1, H)
    gm2 = gamma.reshape(L, 1, H)
    bt2 = beta.reshape(L, 1, H)
    Wo1p = jnp.zeros((H, H), jnp.float32).at[:, : H // 2].set(Wo1)
    bo1p = jnp.zeros((1, H), jnp.float32).at[:, : H // 2].set(bo1)
    Wo2p = jnp.zeros((H, H), jnp.float32).at[: H // 2, :T].set(Wo2)
    bo2p = jnp.zeros((1, H), jnp.float32).at[:, :T].set(bo2)

    degp = _deg_call(col2d, zeros128, ones16)
    h, ms, dinv = _t1_call(xp, W_in, b_in2, Wg[0], degp)
    for l in range(L - 1):
        P = _spmm_call(ms, row2d, col2d, zeros128)
        h, ms = _t2_call(P, h, dinv, bg2[l], gm2[l], bt2[l], Wg[l + 1])
    P = _spmm_call(ms, row2d, col2d, zeros128)
    out = _t3_call(P, h, dinv, bg2[L - 1], gm2[L - 1], bt2[L - 1],
                   Wo1p, bo1p, Wo2p, bo2p)
    return out[:, :T]


# speedup vs baseline: 19.1147x; 1.0092x over previous
import jax
import jax.numpy as jnp
import numpy as np
from jax import lax
from jax.experimental import pallas as pl
from jax.experimental.pallas import tpu as pltpu
from jax.experimental.pallas import tpu_sc as plsc

N = 10000
E = 320000
H = 128
T = 13
L = 6
EPS = 1e-5
BN_SCALE = 1.0 / np.sqrt(1.0 + EPS)

NC = 2
NS = 16
NW = NC * NS
IPC = 128
CPT = 80
EPT = IPC * CPT
EPAD = EPT * NW
NP = 10240
RPT = NP // NS

ROWB = 2048
GRID = NP // ROWB


def _mesh():
    return plsc.VectorSubcoreMesh(
        core_axis_name="c", subcore_axis_name="s", num_cores=NC, num_subcores=NS
    )


DW = 128


def _deg_body(col_hbm, zeros_hbm, ones_hbm, out_hbm, cidx, ones_buf, acc,
              ss0, ss1):
    c = lax.axis_index("c")
    s = lax.axis_index("s")
    ss = (ss0, ss1)

    pltpu.sync_copy(zeros_hbm.at[pl.ds(s * RPT, RPT)],
                    acc.at[pl.ds(s * RPT, RPT)])
    pltpu.sync_copy(ones_hbm, ones_buf)
    tb = (c * NS + s) * CPT
    pltpu.sync_copy(col_hbm.at[pl.ds(tb, CPT)], cidx)
    plsc.subcore_barrier()

    def fire(j, p):
        pltpu.async_copy(ones_buf, acc.at[cidx.at[j]], ss[p], add=True)

    def wait(j, p):
        pltpu.make_async_copy(ones_buf, acc.at[cidx.at[j]], ss[p]).wait()

    fire(0, 0)
    fire(1, 1)

    def body(t, _):
        j = 2 * t
        wait(j, 0)

        @pl.when(j + 2 < CPT)
        def _():
            fire(j + 2, 0)

        wait(j + 1, 1)

        @pl.when(j + 3 < CPT)
        def _():
            fire(j + 3, 1)

        return 0

    lax.fori_loop(0, CPT // 2, body, 0)
    plsc.subcore_barrier()
    pltpu.sync_copy(acc.at[pl.ds(s * RPT, RPT)],
                    out_hbm.at[pl.ds(c * NP + s * RPT, RPT)])


def _deg_call(col2d, zeros16, ones16):
    k = pl.kernel(
        _deg_body,
        out_type=jax.ShapeDtypeStruct((2 * NP, DW), jnp.float32),
        mesh=_mesh(),
        scratch_types=[
            pltpu.VMEM((CPT, IPC), jnp.int32),
            pltpu.VMEM((IPC, DW), jnp.float32),
            pltpu.VMEM_SHARED((NP, DW), jnp.float32),
            pltpu.SemaphoreType.DMA,
            pltpu.SemaphoreType.DMA,
        ],
    )
    return k(col2d, zeros16, ones16)


def _spmm_body(ms_hbm, row_hbm, col_hbm, zeros_hbm, out_hbm,
               cidx, rixr, rb0, rb1, acc,
               si0, si1, sg0, sg1, ss0, ss1):
    c = lax.axis_index("c")
    s = lax.axis_index("s")
    rows = (rb0, rb1)
    si = (si0, si1)
    sg = (sg0, sg1)
    ss = (ss0, ss1)

    @pl.when(c == 0)
    def _():
        pltpu.sync_copy(ms_hbm.at[pl.ds(s * RPT, RPT)],
                        acc.at[pl.ds(s * RPT, RPT)])

    @pl.when(c == 1)
    def _():
        pltpu.sync_copy(zeros_hbm.at[pl.ds(s * RPT, RPT)],
                        acc.at[pl.ds(s * RPT, RPT)])

    tb = (c * NS + s) * CPT
    pltpu.sync_copy(col_hbm.at[pl.ds(tb, CPT)], cidx)
    plsc.subcore_barrier()

    def fire_ridx(j, p):
        pltpu.async_copy(row_hbm.at[tb + j], rixr.at[p], si[p])

    def wait_ridx(j, p):
        pltpu.make_async_copy(row_hbm.at[tb + j], rixr.at[p], si[p]).wait()

    def fire_gather(p):
        pltpu.async_copy(ms_hbm.at[rixr.at[p]], rows[p], sg[p])

    def wait_gather(p):
        pltpu.make_async_copy(ms_hbm.at[rixr.at[p]], rows[p], sg[p]).wait()

    def fire_scatter(j, p):
        pltpu.async_copy(rows[p], acc.at[cidx.at[j]], ss[p], add=True)

    def wait_scatter(j, p):
        pltpu.make_async_copy(rows[p], acc.at[cidx.at[j]], ss[p]).wait()

    fire_ridx(0, 0)
    fire_ridx(1, 1)
    wait_ridx(0, 0)
    fire_gather(0)

    def half(j, p):
        q = 1 - p
        wait_gather(p)
        fire_scatter(j, p)

        @pl.when(j + 2 < CPT)
        def _():
            fire_ridx(j + 2, p)

        @pl.when(j > 0)
        def _():
            wait_scatter(j - 1, q)

        @pl.when(j + 1 < CPT)
        def _():
            wait_ridx(j + 1, q)
            fire_gather(q)

    def body(t, _):
        half(2 * t, 0)
        half(2 * t + 1, 1)
        return 0

    lax.fori_loop(0, CPT // 2, body, 0)
    wait_scatter(CPT - 1, 1)
    plsc.subcore_barrier()
    pltpu.sync_copy(acc.at[pl.ds(s * RPT, RPT)],
                    out_hbm.at[pl.ds(c * NP + s * RPT, RPT)])


def _spmm_call(ms, row2d, col2d, zeros128):
    k = pl.kernel(
        _spmm_body,
        out_type=jax.ShapeDtypeStruct((2 * NP, H), jnp.float32),
        mesh=_mesh(),
        scratch_types=[
            pltpu.VMEM((CPT, IPC), jnp.int32),
            pltpu.VMEM((2, IPC), jnp.int32),
            pltpu.VMEM((IPC, H), jnp.float32),
            pltpu.VMEM((IPC, H), jnp.float32),
            pltpu.VMEM_SHARED((NP, H), jnp.float32),
            pltpu.SemaphoreType.DMA,
            pltpu.SemaphoreType.DMA,
            pltpu.SemaphoreType.DMA,
            pltpu.SemaphoreType.DMA,
            pltpu.SemaphoreType.DMA,
            pltpu.SemaphoreType.DMA,
        ],
    )
    return k(ms, row2d, col2d, zeros128)


def _t1_body(x_ref, win_ref, bin_ref, wg0_ref, d0_ref, d1_ref,
             h0_ref, ms0_ref, dinv_ref):
    deg = 1.0 + d0_ref[:, 0:1] + d1_ref[:, 0:1]
    dv = lax.rsqrt(deg)
    h0 = jnp.dot(x_ref[...], win_ref[...],
                 preferred_element_type=jnp.float32) + bin_ref[...]
    h0_ref[...] = h0
    ms0_ref[...] = jnp.dot(h0, wg0_ref[...],
                           preferred_element_type=jnp.float32) * dv
    dinv_ref[...] = dv


def _t1_call(xp, W_in, b_in2, Wg0, degp):
    return pl.pallas_call(
        _t1_body,
        grid=(GRID,),
        in_specs=[
            pl.BlockSpec((ROWB, H), lambda i: (i, 0)),
            pl.BlockSpec((H, H), lambda i: (0, 0)),
            pl.BlockSpec((1, H), lambda i: (0, 0)),
            pl.BlockSpec((H, H), lambda i: (0, 0)),
            pl.BlockSpec((ROWB, DW), lambda i: (i, 0)),
            pl.BlockSpec((ROWB, DW), lambda i: (GRID + i, 0)),
        ],
        out_specs=[
            pl.BlockSpec((ROWB, H), lambda i: (i, 0)),
            pl.BlockSpec((ROWB, H), lambda i: (i, 0)),
            pl.BlockSpec((ROWB, 1), lambda i: (i, 0)),
        ],
        out_shape=[
            jax.ShapeDtypeStruct((NP, H), jnp.float32),
            jax.ShapeDtypeStruct((NP, H), jnp.float32),
            jax.ShapeDtypeStruct((NP, 1), jnp.float32),
        ],
    )(xp, W_in, b_in2, Wg0, degp, degp)


def _t2_body(p0_ref, p1_ref, h_ref, dinv_ref, bg_ref, gm_ref, bt_ref,
             wn_ref, hn_ref, msn_ref):
    dv = dinv_ref[...]
    agg = dv * (p0_ref[...] + p1_ref[...]) + bg_ref[...]
    hb = agg * BN_SCALE * gm_ref[...] + bt_ref[...]
    hn = jnp.maximum(hb, 0.0) + h_ref[...]
    hn_ref[...] = hn
    msn_ref[...] = jnp.dot(hn, wn_ref[...],
                           preferred_element_type=jnp.float32) * dv


def _t2_call(P, h, dinv, bg2, gm2, bt2, Wnext):
    return pl.pallas_call(
        _t2_body,
        grid=(GRID,),
        in_specs=[
            pl.BlockSpec((ROWB, H), lambda i: (i, 0)),
            pl.BlockSpec((ROWB, H), lambda i: (GRID + i, 0)),
            pl.BlockSpec((ROWB, H), lambda i: (i, 0)),
            pl.BlockSpec((ROWB, 1), lambda i: (i, 0)),
            pl.BlockSpec((1, H), lambda i: (0, 0)),
            pl.BlockSpec((1, H), lambda i: (0, 0)),
            pl.BlockSpec((1, H), lambda i: (0, 0)),
            pl.BlockSpec((H, H), lambda i: (0, 0)),
        ],
        out_specs=[
            pl.BlockSpec((ROWB, H), lambda i: (i, 0)),
            pl.BlockSpec((ROWB, H), lambda i: (i, 0)),
        ],
        out_shape=[
            jax.ShapeDtypeStruct((NP, H), jnp.float32),
            jax.ShapeDtypeStruct((NP, H), jnp.float32),
        ],
    )(P, P, h, dinv, bg2, gm2, bt2, Wnext)


def _t3_body(p0_ref, p1_ref, h_ref, dinv_ref, bg_ref, gm_ref, bt_ref,
             wo1_ref, bo1_ref, wo2_ref, bo2_ref, out_ref, acc):
    i = pl.program_id(0)
    dv = dinv_ref[...]
    agg = dv * (p0_ref[...] + p1_ref[...]) + bg_ref[...]
    hb = agg * BN_SCALE * gm_ref[...] + bt_ref[...]
    hn = jnp.maximum(hb, 0.0) + h_ref[...]
    rid = i * ROWB + lax.broadcasted_iota(jnp.int32, (ROWB, 1), 0)
    hn = jnp.where(rid < N, hn, 0.0)
    bsum = jnp.sum(hn, axis=0, keepdims=True)

    @pl.when(i == 0)
    def _():
        acc[...] = bsum

    @pl.when(i > 0)
    def _():
        acc[...] = acc[...] + bsum

    @pl.when(i == pl.num_programs(0) - 1)
    def _():
        pooled = acc[...] * (1.0 / N)
        o = jnp.maximum(
            jnp.dot(pooled, wo1_ref[...],
                    preferred_element_type=jnp.float32) + bo1_ref[...], 0.0)
        out_ref[...] = jnp.dot(o, wo2_ref[...],
                               preferred_element_type=jnp.float32) + bo2_ref[...]


def _t3_call(P, h, dinv, bg2, gm2, bt2, Wo1p, bo1p, Wo2p, bo2p):
    return pl.pallas_call(
        _t3_body,
        grid=(GRID,),
        in_specs=[
            pl.BlockSpec((ROWB, H), lambda i: (i, 0)),
            pl.BlockSpec((ROWB, H), lambda i: (GRID + i, 0)),
            pl.BlockSpec((ROWB, H), lambda i: (i, 0)),
            pl.BlockSpec((ROWB, 1), lambda i: (i, 0)),
            pl.BlockSpec((1, H), lambda i: (0, 0)),
            pl.BlockSpec((1, H), lambda i: (0, 0)),
            pl.BlockSpec((1, H), lambda i: (0, 0)),
            pl.BlockSpec((H, H), lambda i: (0, 0)),
            pl.BlockSpec((1, H), lambda i: (0, 0)),
            pl.BlockSpec((H, H), lambda i: (0, 0)),
            pl.BlockSpec((1, H), lambda i: (0, 0)),
        ],
        out_specs=pl.BlockSpec((1, H), lambda i: (0, 0)),
        out_shape=jax.ShapeDtypeStruct((1, H), jnp.float32),
        scratch_shapes=[pltpu.VMEM((1, H), jnp.float32)],
    )(P, P, h, dinv, bg2, gm2, bt2, Wo1p, bo1p, Wo2p, bo2p)


def kernel(x, edge_index, W_in, b_in, Wg, bg, gamma, beta, Wo1, bo1, Wo2, bo2):
    pad = EPAD - E
    pidx = jnp.arange(pad, dtype=jnp.int32)
    row = jnp.concatenate([edge_index[0], pidx % np.int32(N)])
    col = jnp.concatenate([edge_index[1], N + pidx % np.int32(NP - N)])
    row2d = row.reshape(EPAD // IPC, IPC)
    col2d = col.reshape(EPAD // IPC, IPC)
    zeros128 = jnp.zeros((NP, H), jnp.float32)
    ones16 = jnp.ones((IPC, DW), jnp.float32)
    xp = jnp.zeros((NP, H), jnp.float32).at[:N].set(x)

    b_in2 = b_in.reshape(1, H)
    bg2 = bg.reshape(L, 1, H)
    gm2 = gamma.reshape(L, 1, H)
    bt2 = beta.reshape(L, 1, H)
    Wo1p = jnp.zeros((H, H), jnp.float32).at[:, : H // 2].set(Wo1)
    bo1p = jnp.zeros((1, H), jnp.float32).at[:, : H // 2].set(bo1)
    Wo2p = jnp.zeros((H, H), jnp.float32).at[: H // 2, :T].set(Wo2)
    bo2p = jnp.zeros((1, H), jnp.float32).at[:, :T].set(bo2)

    degp = _deg_call(col2d, zeros128, ones16)
    h, ms, dinv = _t1_call(xp, W_in, b_in2, Wg[0], degp)
    for l in range(L - 1):
        P = _spmm_call(ms, row2d, col2d, zeros128)
        h, ms = _t2_call(P, h, dinv, bg2[l], gm2[l], bt2[l], Wg[l + 1])
    P = _spmm_call(ms, row2d, col2d, zeros128)
    out = _t3_call(P, h, dinv, bg2[L - 1], gm2[L - 1], bt2[L - 1],
                   Wo1p, bo1p, Wo2p, bo2p)
    return out[:, :T]
